# Initial kernel scaffold; baseline (speedup 1.0000x reference)
import jax
import jax.numpy as jnp
from jax.experimental import pallas as pl


def kernel(protein_node, protein_pos, protein_batch, ligand_node_pert, ligand_pos_pert, ligand_batch, ligand_edge_index, ligand_edge_batch, t, Wp, Wpe, Wl, Wle, Wm1, Wm2, Wn, We, dW1, db1, dW2, db2, dW3, db3):
    raise NotImplementedError("write your pallas kernel here")


# speedup vs baseline: 18.2616x; 18.2616x over previous
import numpy as np
import jax
import jax.numpy as jnp
from jax.experimental import pallas as pl

B = 16; N_LIG = 64; N_PROT = 576; N_PER = 640; K = 16
PNT = 27; LNT = 13; NET = 5
NODE_DIM = 128; EDGE_DIM = 64; TIME_DIM = 16; TMAX = 1000; NL = 2

_OFFS = np.linspace(0.0, float(TMAX), TIME_DIM).astype(np.float32)
_COEFF = float(-0.5 / (_OFFS[1] - _OFFS[0]) ** 2)
_IU, _JU = np.triu_indices(N_LIG, k=1)
NH = _IU.shape[0]


def _silu(x):
    return x * jax.nn.sigmoid(x)


def _main_kernel(lig_x_ref, lig_pos_ref, prot_x_ref, prot_pos_ref, t_ref,
                 Wp_ref, Wl_ref, Wle_ref, Wpe_ref, Wm1_ref, Wm2_ref,
                 Wn_ref, We_ref, hlig_out_ref, elig_out_ref):
    f32 = jnp.float32
    x_l = lig_x_ref[0]
    x_p = prot_x_ref[0]
    t_b = t_ref[pl.program_id(0), 0]
    tn = t_b / float(TMAX)

    offs = (jax.lax.broadcasted_iota(jnp.int32, (1, TIME_DIM), 1)
            .astype(f32) * (float(TMAX) / (TIME_DIM - 1)))
    temb = jnp.exp(_COEFF * (t_b - offs) ** 2)

    h_l = jnp.concatenate([
        x_l @ Wl_ref[...],
        jnp.broadcast_to(temb, (N_LIG, TIME_DIM)),
        jnp.ones((N_LIG, 1), f32),
    ], axis=1)
    h_p = jnp.concatenate([
        x_p @ Wp_ref[...],
        jnp.zeros((N_PROT, 1), f32),
    ], axis=1)
    h = jnp.concatenate([h_l, h_p], axis=0)

    pos = jnp.concatenate([lig_pos_ref[0], prot_pos_ref[0]], axis=0)
    post = pos.T
    dx = pos[:, 0:1] - post[0:1, :]
    dy = pos[:, 1:2] - post[1:2, :]
    dz = pos[:, 2:3] - post[2:3, :]
    d2 = dx * dx + dy * dy + dz * dz

    col = jax.lax.broadcasted_iota(jnp.int32, (N_PER, N_PER), 1)
    row = jax.lax.broadcasted_iota(jnp.int32, (N_PER, N_PER), 0)
    Dm = d2 + jnp.where(row == col, f32(1e9), f32(0.0))

    nbrs = []; vals = []
    for _ in range(K):
        mn = jnp.min(Dm, axis=1, keepdims=True)
        idx = jnp.min(jnp.where(Dm == mn, col, jnp.int32(2**30)),
                      axis=1, keepdims=True)
        nbrs.append(idx); vals.append(mn)
        Dm = jnp.where(col == idx, f32(3e9), Dm)

    ri = jax.lax.broadcasted_iota(jnp.int32, (N_PER, 1), 0)
    lig_i = ri < N_LIG
    km = [jnp.where(lig_i & (idx < N_LIG), f32(0.0), f32(1.0))
          for idx in nbrs]

    Wm1 = Wm1_ref[...]; Wm2 = Wm2_ref[...]
    Wn = Wn_ref[...]; We = We_ref[...]
    dot = lambda a, b: jnp.dot(a, b, preferred_element_type=f32)

    Wpe0 = Wpe_ref[...][0:1, :]
    e_knn = [jnp.broadcast_to(Wpe0, (N_PER, EDGE_DIM))] * K
    Wle = Wle_ref[...]
    A = x_l @ Wle[0:LNT, :]
    Bm = x_l @ Wle[LNT:2 * LNT, :]

    NP2 = N_LIG * N_LIG
    rr = jax.lax.broadcasted_iota(jnp.int32, (NP2, 1), 0)
    ii = rr // N_LIG
    jj = rr - ii * N_LIG
    lane64 = jax.lax.broadcasted_iota(jnp.int32, (NP2, N_LIG), 1)
    R = (lane64 == ii).astype(f32)
    C = (lane64 == jj).astype(f32)
    ndmask = (ii != jj).astype(f32)

    e_lig = jnp.concatenate([
        dot(R, A) + dot(C, Bm),
        jnp.broadcast_to(temb, (NP2, TIME_DIM)),
    ], axis=1)
    posl = pos[0:N_LIG]
    ddl = dot(R, posl) - dot(C, posl)
    d2f = jnp.sum(ddl * ddl, axis=1, keepdims=True)

    for l in range(NL):
        Wm1l = Wm1[l]
        W_s0 = Wm1l[0:128]; W_s1 = Wm1l[128:256]; W_e = Wm1l[256:320]
        w_d2 = Wm1l[320:321]; w_et = Wm1l[321:322]
        Wm2l = Wm2[l]; Wel = We[l]; Wnl = Wn[l]

        Hs0 = dot(h, W_s0)
        Hh1 = dot(h, W_s1)
        base = Hs0 + tn * w_et
        agg = jnp.zeros((N_PER, NODE_DIM), f32)
        new_eknn = []
        for k in range(K):
            oh = (col == nbrs[k]).astype(f32)
            g = dot(oh, Hh1)
            pre = base + g + dot(e_knn[k], W_e) + vals[k] * w_d2
            m2 = _silu(dot(_silu(pre), Wm2l))
            agg = agg + m2 * km[k]
            new_eknn.append(e_knn[k] + _silu(dot(m2, Wel)))
        e_knn = new_eknn

        pre_l = (dot(R, Hs0[0:N_LIG]) + dot(C, Hh1[0:N_LIG])
                 + dot(e_lig, W_e) + d2f * w_d2 + tn * w_et)
        m2l = _silu(dot(_silu(pre_l), Wm2l))
        agg_lig = jax.lax.dot_general(
            R, m2l * ndmask, (((0,), (0,)), ((), ())),
            preferred_element_type=f32)
        agg = agg + jnp.pad(agg_lig, ((0, N_PROT), (0, 0)))
        e_lig = e_lig + _silu(dot(m2l, Wel))

        h = h + _silu(dot(h, Wnl[0:128]) + dot(agg, Wnl[128:256])
                      + tn * Wnl[256:257])

    hlig_out_ref[0] = h[0:N_LIG]
    elig_out_ref[0] = e_lig


def _dec_kernel(ef_ref, eb_ref, hi_ref, hj_ref,
                dW1_ref, db1_ref, dW2_ref, db2_ref, dW3_ref, db3_ref,
                out_ref):
    f32 = jnp.float32
    dot = lambda a, b: jnp.dot(a, b, preferred_element_type=f32)
    dW1 = dW1_ref[...]
    es = ef_ref[0] + eb_ref[0]
    hs = hi_ref[0] + hj_ref[0]
    o = jax.nn.relu(dot(es, dW1[0:EDGE_DIM]) + dot(hs, dW1[EDGE_DIM:])
                    + db1_ref[...])
    o = jax.nn.relu(dot(o, dW2_ref[...]) + db2_ref[...])
    out_ref[0] = dot(o, dW3_ref[...]) + db3_ref[...]


def kernel(protein_node, protein_pos, protein_batch, ligand_node_pert,
           ligand_pos_pert, ligand_batch, ligand_edge_index,
           ligand_edge_batch, t, Wp, Wpe, Wl, Wle, Wm1, Wm2, Wn, We,
           dW1, db1, dW2, db2, dW3, db3):
    f32 = jnp.float32
    lig_x = ligand_node_pert.reshape(B, N_LIG, LNT)
    lig_pos = ligand_pos_pert.reshape(B, N_LIG, 3).astype(f32)
    prot_x = protein_node.reshape(B, N_PROT, PNT)
    prot_pos = protein_pos.reshape(B, N_PROT, 3).astype(f32)
    t_f = t.astype(f32).reshape(B, 1)

    def bspec(shape):
        nd = len(shape)
        return pl.BlockSpec((1,) + shape, lambda b: (b,) + (0,) * nd)

    def wspec(shape):
        nd = len(shape)
        return pl.BlockSpec(shape, lambda b: (0,) * nd)

    hlig, elig = pl.pallas_call(
        _main_kernel,
        grid=(B,),
        in_specs=[
            bspec((N_LIG, LNT)), bspec((N_LIG, 3)),
            bspec((N_PROT, PNT)), bspec((N_PROT, 3)),
            pl.BlockSpec((B, 1), lambda b: (0, 0)),
            wspec((PNT, NODE_DIM - 1)), wspec((LNT, 111)),
            wspec((2 * LNT, EDGE_DIM - TIME_DIM)), wspec((NET, EDGE_DIM)),
            wspec((NL, 322, NODE_DIM)), wspec((NL, NODE_DIM, NODE_DIM)),
            wspec((NL, 257, NODE_DIM)), wspec((NL, NODE_DIM, EDGE_DIM)),
        ],
        out_specs=[bspec((N_LIG, NODE_DIM)),
                   bspec((N_LIG * N_LIG, EDGE_DIM))],
        out_shape=[
            jax.ShapeDtypeStruct((B, N_LIG, NODE_DIM), f32),
            jax.ShapeDtypeStruct((B, N_LIG * N_LIG, EDGE_DIM), f32),
        ],
    )(lig_x, lig_pos, prot_x, prot_pos, t_f,
      Wp, Wl, Wle, Wpe, Wm1, Wm2, Wn, We)

    e3 = elig.reshape(B, N_LIG, N_LIG, EDGE_DIM)
    iu = jnp.asarray(_IU); ju = jnp.asarray(_JU)
    ef = e3[:, iu, ju]; eb = e3[:, ju, iu]
    hi = hlig[:, iu]; hj = hlig[:, ju]

    out = pl.pallas_call(
        _dec_kernel,
        grid=(B,),
        in_specs=[
            bspec((NH, EDGE_DIM)), bspec((NH, EDGE_DIM)),
            bspec((NH, NODE_DIM)), bspec((NH, NODE_DIM)),
            wspec((NODE_DIM + EDGE_DIM, EDGE_DIM)), wspec((1, EDGE_DIM)),
            wspec((EDGE_DIM, EDGE_DIM)), wspec((1, EDGE_DIM)),
            wspec((EDGE_DIM, NET)), wspec((1, NET)),
        ],
        out_specs=[bspec((NH, NET))],
        out_shape=[jax.ShapeDtypeStruct((B, NH, NET), f32)],
    )(ef, eb, hi, hj, dW1, db1.reshape(1, -1), dW2, db2.reshape(1, -1),
      dW3, db3.reshape(1, -1))[0]

    return out.reshape(B * NH, NET)
